# Initial kernel scaffold; baseline (speedup 1.0000x reference)
import jax
import jax.numpy as jnp
from jax.experimental import pallas as pl


def kernel(x, edge_index, W, b):
    raise NotImplementedError("write your pallas kernel here")


# speedup vs baseline: 1.2914x; 1.2914x over previous
import functools

import jax
import jax.numpy as jnp
from jax import lax
from jax.experimental import pallas as pl
from jax.experimental.pallas import tpu as pltpu
from jax.experimental.pallas import tpu_sc as plsc

N = 10000
E = 160000
D = 256
OUT = 256

NC = 2
NS = 16
NW = NC * NS
NPG = 160
NGRP = 2
NPAD = NW * NGRP * NPG
DUMP = NPG

CHUNK = 6400
NCHUNK = E // CHUNK
BATCH = 1
NBATCH = CHUNK // (16 * BATCH)
GB = 32
BIG = 1 << 10

_mesh = plsc.VectorSubcoreMesh(core_axis_name="c", subcore_axis_name="s")


def _treemin(v, iota16):
    for sh in (1, 2, 4, 8):
        v = jnp.minimum(v, jnp.take(v, (iota16 + sh) % 16, mode="wrap"))
    return v


def _treesum(v, iota16):
    for sh in (1, 2, 4, 8):
        v = v + jnp.take(v, (iota16 + sh) % 16, mode="wrap")
    return v


@functools.partial(
    pl.kernel,
    out_type=[
        jax.ShapeDtypeStruct((NPAD * D,), jnp.float32),
        jax.ShapeDtypeStruct((NPAD * D,), jnp.float32),
        jax.ShapeDtypeStruct((NPAD,), jnp.float32),
    ],
    mesh=_mesh,
    scratch_types=[
        pltpu.VMEM((CHUNK,), jnp.int32),
        pltpu.VMEM((CHUNK,), jnp.int32),
        pltpu.VMEM((CHUNK + GB,), jnp.int32),
        pltpu.VMEM((CHUNK + GB,), jnp.int32),
        pltpu.VMEM((GB,), jnp.int32),
        pltpu.VMEM((GB, D), jnp.float32),
        pltpu.VMEM(((NPG + 1) * D,), jnp.float32),
        pltpu.VMEM(((NPG + 1) * D,), jnp.float32),
        pltpu.VMEM((NPG + 16,), jnp.float32),
        pltpu.SMEM((NPG + 1,), jnp.float32),
    ],
)
def _sc_segment_reduce(x_hbm, src_hbm, dst_hbm, ssum_hbm, smax_hbm, deg_hbm,
                       dstb, srcb, msrc, mdstl, subsrc, rows,
                       accs, accm, degv, degs):
    wid = lax.axis_index("s") * NC + lax.axis_index("c")
    iota16 = jnp.arange(16, dtype=jnp.int32)
    zero16 = jnp.zeros((16,), jnp.float32)
    ninf16 = jnp.full((16,), -jnp.inf, jnp.float32)
    dump16 = jnp.full((16,), DUMP, jnp.int32)
    zero16i = jnp.zeros((16,), jnp.int32)

    for g in range(NGRP):
        lo = (wid * NGRP + g) * NPG
        hi = lo + NPG

        def zacc(i, carry):
            accs[pl.ds(i * 16, 16)] = zero16
            accm[pl.ds(i * 16, 16)] = ninf16
            return carry

        lax.fori_loop(0, (NPG + 1) * D // 16, zacc, 0)

        def zdeg(i, carry):
            degs[i] = 0.0
            return carry

        lax.fori_loop(0, NPG + 1, zdeg, 0)

        def chunk_body(ci, carry):
            base = ci * CHUNK
            pltpu.sync_copy(dst_hbm.at[pl.ds(base, CHUNK)], dstb)
            pltpu.sync_copy(src_hbm.at[pl.ds(base, CHUNK)], srcb)

            def scan_body(bi, cnt):
                boff = bi * (16 * BATCH)
                msel = None
                ktot = None
                for j in range(BATCH):
                    d = dstb[pl.ds(boff + j * 16, 16)]
                    m = (d >= lo) & (d < hi)
                    mj = jnp.where(m, iota16 + j * 16, BIG)
                    kj = jnp.where(m, 1, 0)
                    msel = mj if msel is None else jnp.minimum(msel, mj)
                    ktot = kj if ktot is None else ktot + kj
                k = jnp.take(_treesum(ktot, iota16), iota16, mode="wrap")[0]

                def match_body(_, carry2):
                    cnt2, ms = carry2
                    mnv = _treemin(ms, iota16)
                    mnd = jnp.take(mnv, iota16, mode="wrap")
                    mn = mnd[0]
                    j = mn // 16
                    voff = boff + j * 16
                    sv = srcb[pl.ds(voff, 16)]
                    dv = dstb[pl.ds(voff, 16)]
                    ssp = jnp.take(sv, mnd, mode="wrap")
                    dsp = jnp.take(dv, mnd, mode="wrap")
                    msrc[pl.ds(cnt2, 16)] = ssp
                    mdstl[pl.ds(cnt2, 16)] = dsp - lo
                    ms = jnp.where(ms == mnv, BIG, ms)
                    return (cnt2 + 1, ms)

                cnt, _ = lax.fori_loop(0, k, match_body, (cnt, msel))
                return cnt

            cnt = lax.fori_loop(0, NBATCH, scan_body, 0)

            for i in range(GB // 16):
                msrc[pl.ds(cnt + i * 16, 16)] = iota16
                mdstl[pl.ds(cnt + i * 16, 16)] = dump16

            nsub = (cnt + GB - 1) // GB

            def sub_body(si, carry2):
                sb = si * GB
                for i in range(GB // 16):
                    subsrc[pl.ds(i * 16, 16)] = msrc[pl.ds(sb + i * 16, 16)]
                pltpu.sync_copy(x_hbm.at[subsrc], rows)

                def row_body(r, carry3):
                    rdiv = r // 16
                    rm = r - rdiv * 16
                    dv = mdstl[pl.ds(sb + rdiv * 16, 16)]
                    dsp = jnp.take(dv, iota16 + rm, mode="wrap")
                    dstl = dsp[0]
                    rbase = dstl * D
                    for c in range(D // 16):
                        rv = rows[r, pl.ds(c * 16, 16)]
                        cs = accs[pl.ds(rbase + c * 16, 16)]
                        accs[pl.ds(rbase + c * 16, 16)] = cs + rv
                        cm = accm[pl.ds(rbase + c * 16, 16)]
                        accm[pl.ds(rbase + c * 16, 16)] = jnp.maximum(cm, rv)
                    degs[dstl] = degs[dstl] + 1.0
                    return carry3

                lax.fori_loop(0, GB, row_body, 0)
                return carry2

            lax.fori_loop(0, nsub, sub_body, 0)
            return carry

        lax.fori_loop(0, NCHUNK, chunk_body, 0)

        def dstage(i, carry):
            degv[pl.ds(i, 16)] = zero16 + degs[i]
            return carry

        lax.fori_loop(0, NPG, dstage, 0)

        pltpu.sync_copy(accs.at[pl.ds(0, NPG * D)],
                        ssum_hbm.at[pl.ds(lo * D, NPG * D)])
        pltpu.sync_copy(accm.at[pl.ds(0, NPG * D)],
                        smax_hbm.at[pl.ds(lo * D, NPG * D)])
        pltpu.sync_copy(degv.at[pl.ds(0, NPG)], deg_hbm.at[pl.ds(lo, NPG)])


BN = 400


def _tc_body(x_ref, ssum_ref, smax_ref, deg_ref, w1_ref, w2_ref, w3_ref,
             b_ref, out_ref):
    deg = deg_ref[...]
    rdeg = 1.0 / jnp.maximum(deg, 1.0)
    mean = ssum_ref[...] * rdeg
    mx = jnp.where(deg > 0, smax_ref[...], 0.0)
    acc = jnp.dot(x_ref[...], w1_ref[...], preferred_element_type=jnp.float32)
    acc += jnp.dot(mean, w2_ref[...], preferred_element_type=jnp.float32)
    acc += jnp.dot(mx, w3_ref[...], preferred_element_type=jnp.float32)
    out_ref[...] = jnp.maximum(acc + b_ref[...], 0.0)


_tc_linear = pl.pallas_call(
    _tc_body,
    grid=(N // BN,),
    in_specs=[
        pl.BlockSpec((BN, D), lambda i: (i, 0)),
        pl.BlockSpec((BN, D), lambda i: (i, 0)),
        pl.BlockSpec((BN, D), lambda i: (i, 0)),
        pl.BlockSpec((BN, 1), lambda i: (i, 0)),
        pl.BlockSpec((D, OUT), lambda i: (0, 0)),
        pl.BlockSpec((D, OUT), lambda i: (0, 0)),
        pl.BlockSpec((D, OUT), lambda i: (0, 0)),
        pl.BlockSpec((1, OUT), lambda i: (0, 0)),
    ],
    out_specs=pl.BlockSpec((BN, OUT), lambda i: (i, 0)),
    out_shape=jax.ShapeDtypeStruct((N, OUT), jnp.float32),
)


def kernel(x, edge_index, W, b):
    src = edge_index[0]
    dst = edge_index[1]
    ssum_f, smax_f, deg = _sc_segment_reduce(x, src, dst)
    ssum = ssum_f.reshape(NPAD, D)[:N]
    smax = smax_f.reshape(NPAD, D)[:N]
    wt = W.T
    return _tc_linear(
        x, ssum, smax, deg[:N].reshape(N, 1),
        wt[:D], wt[D:2 * D], wt[2 * D:], b.reshape(1, OUT))


# speedup vs baseline: 1.3408x; 1.0382x over previous
import functools

import jax
import jax.numpy as jnp
from jax import lax
from jax.experimental import pallas as pl
from jax.experimental.pallas import tpu as pltpu
from jax.experimental.pallas import tpu_sc as plsc

N = 10000
E = 160000
D = 256
DH = 128
OUT = 256

NC = 2
NS = 16
NW = NC * NS
NPG = 160
NGRP = 2
NPAD = NW * NGRP * NPG
DUMP = NPG
SSTR = NPG + 8

CHUNK = 6400
NCHUNK = E // CHUNK
NVREG = CHUNK // 16
GB = 32
BIG = 1 << 10

_mesh = plsc.VectorSubcoreMesh(core_axis_name="c", subcore_axis_name="s")


def _treemin(v, iota16):
    for sh in (1, 2, 4, 8):
        v = jnp.minimum(v, jnp.take(v, (iota16 + sh) % 16, mode="wrap"))
    return v


def _treesum(v, iota16):
    for sh in (1, 2, 4, 8):
        v = v + jnp.take(v, (iota16 + sh) % 16, mode="wrap")
    return v


@functools.partial(
    pl.kernel,
    out_type=[
        jax.ShapeDtypeStruct((NPAD, DH), jnp.float32),
        jax.ShapeDtypeStruct((NPAD, DH), jnp.float32),
        jax.ShapeDtypeStruct((NPAD, D), jnp.float32),
        jax.ShapeDtypeStruct((NPAD,), jnp.float32),
    ],
    mesh=_mesh,
    scratch_types=[
        pltpu.VMEM((CHUNK,), jnp.int32),
        pltpu.VMEM((CHUNK,), jnp.int32),
        pltpu.VMEM((CHUNK + 3 * GB,), jnp.int32),
        pltpu.VMEM((CHUNK + 3 * GB,), jnp.int32),
        pltpu.VMEM((GB,), jnp.int32),
        pltpu.VMEM((GB,), jnp.int32),
        pltpu.VMEM((GB,), jnp.int32),
        pltpu.VMEM((GB,), jnp.int32),
        pltpu.VMEM((GB, DH), jnp.float32),
        pltpu.VMEM((GB, DH), jnp.float32),
        pltpu.VMEM((GB, DH), jnp.float32),
        pltpu.VMEM((GB, DH), jnp.float32),
        pltpu.VMEM((NPG + 1, D), jnp.float32),
        pltpu.VMEM((NPG + 16,), jnp.float32),
        pltpu.SMEM((NPG + 1,), jnp.float32),
        pltpu.VMEM_SHARED((NS * SSTR, DH), jnp.float32),
        pltpu.VMEM_SHARED((NS * SSTR, DH), jnp.float32),
        pltpu.SemaphoreType.DMA,
        pltpu.SemaphoreType.DMA,
    ],
)
def _sc_segment_reduce(xlo_hbm, xhi_hbm, src_hbm, dst_hbm,
                       slo_hbm, shi_hbm, smax_hbm, deg_hbm,
                       dstb, srcb, msrc, mdstl, gsrc_a, gsrc_b, gidx_a, gidx_b,
                       rlo_a, rhi_a, rlo_b, rhi_b, accm, degv, degs,
                       shlo, shhi, sem_a, sem_b):
    sid = lax.axis_index("s")
    wid = sid * NC + lax.axis_index("c")
    iota16 = jnp.arange(16, dtype=jnp.int32)
    zero16 = jnp.zeros((16,), jnp.float32)
    ninf16 = jnp.full((16,), -jnp.inf, jnp.float32)
    dump16 = jnp.full((16,), DUMP, jnp.int32)
    srow = sid * SSTR

    for g in range(NGRP):
        lo = (wid * NGRP + g) * NPG
        hi = lo + NPG

        def zrows(i, carry):
            rlo_a[i // 8, pl.ds((i % 8) * 16, 16)] = zero16
            return carry

        lax.fori_loop(0, GB * DH // 16, zrows, 0)
        for kk in range(NPG // GB):
            pltpu.sync_copy(rlo_a, shlo.at[pl.ds(srow + kk * GB, GB)])
            pltpu.sync_copy(rlo_a, shhi.at[pl.ds(srow + kk * GB, GB)])
        pltpu.sync_copy(rlo_a.at[pl.ds(0, 8)], shlo.at[pl.ds(srow + NPG, 8)])
        pltpu.sync_copy(rlo_a.at[pl.ds(0, 8)], shhi.at[pl.ds(srow + NPG, 8)])

        def zacc(i, carry):
            for c in range(D // 16):
                accm[i, pl.ds(c * 16, 16)] = ninf16
            return carry

        lax.fori_loop(0, NPG + 1, zacc, 0)

        def zdeg(i, carry):
            degs[i] = 0.0
            return carry

        lax.fori_loop(0, NPG + 1, zdeg, 0)

        def chunk_body(ci, carry):
            base = ci * CHUNK
            pltpu.sync_copy(dst_hbm.at[pl.ds(base, CHUNK)], dstb)
            pltpu.sync_copy(src_hbm.at[pl.ds(base, CHUNK)], srcb)

            def scan_body(bi, cnt):
                boff = bi * 16
                d = dstb[pl.ds(boff, 16)]
                m = (d >= lo) & (d < hi)
                msel = jnp.where(m, iota16, BIG)
                ktot = jnp.where(m, 1, 0)
                k = jnp.take(_treesum(ktot, iota16), iota16, mode="wrap")[0]

                def match_body(_, carry2):
                    cnt2, ms = carry2
                    mnv = _treemin(ms, iota16)
                    mnd = jnp.take(mnv, iota16, mode="wrap")
                    sv = srcb[pl.ds(boff, 16)]
                    ssp = jnp.take(sv, mnd, mode="wrap")
                    dsp = jnp.take(d, mnd, mode="wrap")
                    msrc[pl.ds(cnt2, 16)] = ssp
                    mdstl[pl.ds(cnt2, 16)] = dsp - lo
                    ms = jnp.where(ms == mnv, BIG, ms)
                    return (cnt2 + 1, ms)

                cnt, _ = lax.fori_loop(0, k, match_body, (cnt, msel))
                return cnt

            cnt = lax.fori_loop(0, NVREG, scan_body, 0)

            for i in range(3 * GB // 16):
                msrc[pl.ds(cnt + i * 16, 16)] = iota16
                mdstl[pl.ds(cnt + i * 16, 16)] = dump16

            pairs = jnp.maximum((cnt + 2 * GB - 1) // (2 * GB), 1)

            def build_and_start(si, gsrc, gidx, rlo, rhi, sem):
                sb = si * GB
                for i in range(GB // 16):
                    gsrc[pl.ds(i * 16, 16)] = msrc[pl.ds(sb + i * 16, 16)]
                    gidx[pl.ds(i * 16, 16)] = (
                        mdstl[pl.ds(sb + i * 16, 16)] + srow)
                pltpu.async_copy(xlo_hbm.at[gsrc], rlo, sem)
                pltpu.async_copy(xhi_hbm.at[gsrc], rhi, sem)

            def phase(si, gsrc, gidx, rlo, rhi, sem,
                      ngsrc, ngidx, nrlo, nrhi, nsem):
                pltpu.make_async_copy(xlo_hbm.at[gsrc], rlo, sem).wait()
                pltpu.make_async_copy(xhi_hbm.at[gsrc], rhi, sem).wait()
                build_and_start(si + 1, ngsrc, ngidx, nrlo, nrhi, nsem)
                pltpu.sync_copy(rlo, shlo.at[gidx], add=True)
                pltpu.sync_copy(rhi, shhi.at[gidx], add=True)
                sb = si * GB

                def row_body(r, carry3):
                    rdiv = r // 16
                    rm = r - rdiv * 16
                    dv = mdstl[pl.ds(sb + rdiv * 16, 16)]
                    dsp = jnp.take(dv, iota16 + rm, mode="wrap")
                    dstl = dsp[0]
                    for c in range(DH // 16):
                        rv = rlo[r, pl.ds(c * 16, 16)]
                        cm = accm[dstl, pl.ds(c * 16, 16)]
                        accm[dstl, pl.ds(c * 16, 16)] = jnp.maximum(cm, rv)
                    for c in range(DH // 16):
                        rv = rhi[r, pl.ds(c * 16, 16)]
                        cm = accm[dstl, pl.ds(DH + c * 16, 16)]
                        accm[dstl, pl.ds(DH + c * 16, 16)] = (
                            jnp.maximum(cm, rv))
                    degs[dstl] = degs[dstl] + 1.0
                    return carry3

                lax.fori_loop(0, GB, row_body, 0)

            build_and_start(0, gsrc_a, gidx_a, rlo_a, rhi_a, sem_a)

            def pair_body(p, carry2):
                phase(2 * p, gsrc_a, gidx_a, rlo_a, rhi_a, sem_a,
                      gsrc_b, gidx_b, rlo_b, rhi_b, sem_b)
                phase(2 * p + 1, gsrc_b, gidx_b, rlo_b, rhi_b, sem_b,
                      gsrc_a, gidx_a, rlo_a, rhi_a, sem_a)
                return carry2

            lax.fori_loop(0, pairs, pair_body, 0)
            pltpu.make_async_copy(xlo_hbm.at[gsrc_a], rlo_a, sem_a).wait()
            pltpu.make_async_copy(xhi_hbm.at[gsrc_a], rhi_a, sem_a).wait()
            return carry

        lax.fori_loop(0, NCHUNK, chunk_body, 0)

        def dstage(i, carry):
            degv[pl.ds(i, 16)] = zero16 + degs[i]
            return carry

        lax.fori_loop(0, NPG, dstage, 0)

        pltpu.sync_copy(shlo.at[pl.ds(srow, NPG)], slo_hbm.at[pl.ds(lo, NPG)])
        pltpu.sync_copy(shhi.at[pl.ds(srow, NPG)], shi_hbm.at[pl.ds(lo, NPG)])
        pltpu.sync_copy(accm.at[pl.ds(0, NPG)], smax_hbm.at[pl.ds(lo, NPG)])
        pltpu.sync_copy(degv.at[pl.ds(0, NPG)], deg_hbm.at[pl.ds(lo, NPG)])


BN = 400


def _tc_body(x_ref, slo_ref, shi_ref, smax_ref, deg_ref,
             w1_ref, w2a_ref, w2b_ref, w3_ref, b_ref, out_ref):
    deg = deg_ref[...]
    rdeg = 1.0 / jnp.maximum(deg, 1.0)
    mx = jnp.where(deg > 0, smax_ref[...], 0.0)
    acc = jnp.dot(x_ref[...], w1_ref[...], preferred_element_type=jnp.float32)
    acc += jnp.dot(slo_ref[...] * rdeg, w2a_ref[...],
                   preferred_element_type=jnp.float32)
    acc += jnp.dot(shi_ref[...] * rdeg, w2b_ref[...],
                   preferred_element_type=jnp.float32)
    acc += jnp.dot(mx, w3_ref[...], preferred_element_type=jnp.float32)
    out_ref[...] = jnp.maximum(acc + b_ref[...], 0.0)


_tc_linear = pl.pallas_call(
    _tc_body,
    grid=(N // BN,),
    in_specs=[
        pl.BlockSpec((BN, D), lambda i: (i, 0)),
        pl.BlockSpec((BN, DH), lambda i: (i, 0)),
        pl.BlockSpec((BN, DH), lambda i: (i, 0)),
        pl.BlockSpec((BN, D), lambda i: (i, 0)),
        pl.BlockSpec((BN, 1), lambda i: (i, 0)),
        pl.BlockSpec((D, OUT), lambda i: (0, 0)),
        pl.BlockSpec((DH, OUT), lambda i: (0, 0)),
        pl.BlockSpec((DH, OUT), lambda i: (0, 0)),
        pl.BlockSpec((D, OUT), lambda i: (0, 0)),
        pl.BlockSpec((1, OUT), lambda i: (0, 0)),
    ],
    out_specs=pl.BlockSpec((BN, OUT), lambda i: (i, 0)),
    out_shape=jax.ShapeDtypeStruct((N, OUT), jnp.float32),
)


def kernel(x, edge_index, W, b):
    src = edge_index[0]
    dst = edge_index[1]
    xlo = x[:, :DH]
    xhi = x[:, DH:]
    slo, shi, smax, deg = _sc_segment_reduce(xlo, xhi, src, dst)
    wt = W.T
    return _tc_linear(
        x, slo[:N], shi[:N], smax[:N], deg[:N].reshape(N, 1),
        wt[:D], wt[D:D + DH], wt[D + DH:2 * D], wt[2 * D:], b.reshape(1, OUT))


# speedup vs baseline: 1.7862x; 1.3322x over previous
import functools

import jax
import jax.numpy as jnp
from jax import lax
from jax.experimental import pallas as pl
from jax.experimental.pallas import tpu as pltpu
from jax.experimental.pallas import tpu_sc as plsc

N = 10000
E = 160000
D = 256
DH = 128
OUT = 256

NC = 2
NS = 16
NW = NC * NS
NPG = 160
NGRP = 2
NPAD = NW * NGRP * NPG
DUMP = NPG
SSTR = NPG + 8

CHUNK = 6400
NCHUNK = E // CHUNK
NVREG = CHUNK // 16
GB = 32
BIG = 1 << 10

_mesh = plsc.VectorSubcoreMesh(core_axis_name="c", subcore_axis_name="s")


def _treemin(v, iota16):
    for sh in (1, 2, 4, 8):
        v = jnp.minimum(v, jnp.take(v, (iota16 + sh) % 16, mode="wrap"))
    return v


def _treesum(v, iota16):
    for sh in (1, 2, 4, 8):
        v = v + jnp.take(v, (iota16 + sh) % 16, mode="wrap")
    return v


@functools.partial(
    pl.kernel,
    out_type=[
        jax.ShapeDtypeStruct((NPAD, DH), jnp.float32),
        jax.ShapeDtypeStruct((NPAD, DH), jnp.float32),
        jax.ShapeDtypeStruct((NPAD, D), jnp.float32),
        jax.ShapeDtypeStruct((NPAD,), jnp.float32),
    ],
    mesh=_mesh,
    scratch_types=[
        pltpu.VMEM((CHUNK,), jnp.int32),
        pltpu.VMEM((CHUNK,), jnp.int32),
        pltpu.VMEM((CHUNK + 3 * GB,), jnp.int32),
        pltpu.VMEM((CHUNK + 3 * GB,), jnp.int32),
        pltpu.VMEM((GB,), jnp.int32),
        pltpu.VMEM((GB,), jnp.int32),
        pltpu.VMEM((GB,), jnp.int32),
        pltpu.VMEM((GB,), jnp.int32),
        pltpu.VMEM((GB, DH), jnp.float32),
        pltpu.VMEM((GB, DH), jnp.float32),
        pltpu.VMEM((GB, DH), jnp.float32),
        pltpu.VMEM((GB, DH), jnp.float32),
        pltpu.VMEM((NPG + 1, D), jnp.float32),
        pltpu.VMEM((NPG + 16,), jnp.float32),
        pltpu.SMEM((NPG + 1,), jnp.float32),
        pltpu.VMEM_SHARED((NS * SSTR, DH), jnp.float32),
        pltpu.VMEM_SHARED((NS * SSTR, DH), jnp.float32),
        pltpu.SemaphoreType.DMA,
        pltpu.SemaphoreType.DMA,
    ],
)
def _sc_segment_reduce(xlo_hbm, xhi_hbm, src_hbm, dst_hbm,
                       slo_hbm, shi_hbm, smax_hbm, deg_hbm,
                       dstb, srcb, msrc, mdstl, gsrc_a, gsrc_b, gidx_a, gidx_b,
                       rlo_a, rhi_a, rlo_b, rhi_b, accm, degv, degs,
                       shlo, shhi, sem_a, sem_b):
    sid = lax.axis_index("s")
    wid = sid * NC + lax.axis_index("c")
    iota16 = jnp.arange(16, dtype=jnp.int32)
    zero16 = jnp.zeros((16,), jnp.float32)
    ninf16 = jnp.full((16,), -jnp.inf, jnp.float32)
    dump16 = jnp.full((16,), DUMP, jnp.int32)
    srow = sid * SSTR

    for g in range(NGRP):
        lo = (wid * NGRP + g) * NPG
        hi = lo + NPG

        def zrows(i, carry):
            rlo_a[i // 8, pl.ds((i % 8) * 16, 16)] = zero16
            return carry

        lax.fori_loop(0, GB * DH // 16, zrows, 0)
        for kk in range(NPG // GB):
            pltpu.sync_copy(rlo_a, shlo.at[pl.ds(srow + kk * GB, GB)])
            pltpu.sync_copy(rlo_a, shhi.at[pl.ds(srow + kk * GB, GB)])
        pltpu.sync_copy(rlo_a.at[pl.ds(0, 8)], shlo.at[pl.ds(srow + NPG, 8)])
        pltpu.sync_copy(rlo_a.at[pl.ds(0, 8)], shhi.at[pl.ds(srow + NPG, 8)])

        def zacc(i, carry):
            for c in range(D // 16):
                accm[i, pl.ds(c * 16, 16)] = ninf16
            return carry

        lax.fori_loop(0, NPG + 1, zacc, 0)

        def zdeg(i, carry):
            degs[i] = 0.0
            return carry

        lax.fori_loop(0, NPG + 1, zdeg, 0)

        def chunk_body(ci, carry):
            base = ci * CHUNK
            pltpu.sync_copy(dst_hbm.at[pl.ds(base, CHUNK)], dstb)
            pltpu.sync_copy(src_hbm.at[pl.ds(base, CHUNK)], srcb)

            def scan_body(bi, cnt):
                boff = bi * 64
                msels = []
                ktot = None
                for j in range(4):
                    d = dstb[pl.ds(boff + j * 16, 16)]
                    m = (d >= lo) & (d < hi)
                    msels.append(jnp.where(m, iota16 + j * 16, BIG))
                    kj = jnp.where(m, 1, 0)
                    ktot = kj if ktot is None else ktot + kj
                k = jnp.take(_treesum(ktot, iota16), iota16, mode="wrap")[0]

                def match_body(_, carry2):
                    cnt2, m0, m1, m2, m3 = carry2
                    comb = jnp.minimum(jnp.minimum(m0, m1),
                                       jnp.minimum(m2, m3))
                    mnv = _treemin(comb, iota16)
                    mnd = jnp.take(mnv, iota16, mode="wrap")
                    j = mnd[0] // 16
                    voff = boff + j * 16
                    sv = srcb[pl.ds(voff, 16)]
                    dv = dstb[pl.ds(voff, 16)]
                    ssp = jnp.take(sv, mnd, mode="wrap")
                    dsp = jnp.take(dv, mnd, mode="wrap")
                    msrc[pl.ds(cnt2, 16)] = ssp
                    mdstl[pl.ds(cnt2, 16)] = dsp - lo
                    m0 = jnp.where(m0 == mnv, BIG, m0)
                    m1 = jnp.where(m1 == mnv, BIG, m1)
                    m2 = jnp.where(m2 == mnv, BIG, m2)
                    m3 = jnp.where(m3 == mnv, BIG, m3)
                    return (cnt2 + 1, m0, m1, m2, m3)

                out = lax.fori_loop(0, k, match_body, (cnt, *msels))
                return out[0]

            cnt = lax.fori_loop(0, NVREG // 4, scan_body, 0)

            for i in range(3 * GB // 16):
                msrc[pl.ds(cnt + i * 16, 16)] = iota16
                mdstl[pl.ds(cnt + i * 16, 16)] = dump16

            pairs = jnp.maximum((cnt + 2 * GB - 1) // (2 * GB), 1)

            def build_and_start(si, gsrc, gidx, rlo, rhi, sem):
                sb = si * GB
                for i in range(GB // 16):
                    gsrc[pl.ds(i * 16, 16)] = msrc[pl.ds(sb + i * 16, 16)]
                    gidx[pl.ds(i * 16, 16)] = (
                        mdstl[pl.ds(sb + i * 16, 16)] + srow)
                pltpu.async_copy(xlo_hbm.at[gsrc], rlo, sem)
                pltpu.async_copy(xhi_hbm.at[gsrc], rhi, sem)

            def phase(si, gsrc, gidx, rlo, rhi, sem,
                      ngsrc, ngidx, nrlo, nrhi, nsem):
                pltpu.make_async_copy(xlo_hbm.at[gsrc], rlo, sem).wait()
                pltpu.make_async_copy(xhi_hbm.at[gsrc], rhi, sem).wait()
                build_and_start(si + 1, ngsrc, ngidx, nrlo, nrhi, nsem)
                pltpu.sync_copy(rlo, shlo.at[gidx], add=True)
                pltpu.sync_copy(rhi, shhi.at[gidx], add=True)
                sb = si * GB

                def row_body(r, carry3):
                    rdiv = r // 16
                    rm = r - rdiv * 16
                    dv = mdstl[pl.ds(sb + rdiv * 16, 16)]
                    dsp = jnp.take(dv, iota16 + rm, mode="wrap")
                    dstl = dsp[0]
                    for c in range(DH // 16):
                        rv = rlo[r, pl.ds(c * 16, 16)]
                        cm = accm[dstl, pl.ds(c * 16, 16)]
                        accm[dstl, pl.ds(c * 16, 16)] = jnp.maximum(cm, rv)
                    for c in range(DH // 16):
                        rv = rhi[r, pl.ds(c * 16, 16)]
                        cm = accm[dstl, pl.ds(DH + c * 16, 16)]
                        accm[dstl, pl.ds(DH + c * 16, 16)] = (
                            jnp.maximum(cm, rv))
                    degs[dstl] = degs[dstl] + 1.0
                    return carry3

                lax.fori_loop(0, GB, row_body, 0)

            build_and_start(0, gsrc_a, gidx_a, rlo_a, rhi_a, sem_a)

            def pair_body(p, carry2):
                phase(2 * p, gsrc_a, gidx_a, rlo_a, rhi_a, sem_a,
                      gsrc_b, gidx_b, rlo_b, rhi_b, sem_b)
                phase(2 * p + 1, gsrc_b, gidx_b, rlo_b, rhi_b, sem_b,
                      gsrc_a, gidx_a, rlo_a, rhi_a, sem_a)
                return carry2

            lax.fori_loop(0, pairs, pair_body, 0)
            pltpu.make_async_copy(xlo_hbm.at[gsrc_a], rlo_a, sem_a).wait()
            pltpu.make_async_copy(xhi_hbm.at[gsrc_a], rhi_a, sem_a).wait()
            return carry

        lax.fori_loop(0, NCHUNK, chunk_body, 0)

        def dstage(i, carry):
            degv[pl.ds(i, 16)] = zero16 + degs[i]
            return carry

        lax.fori_loop(0, NPG, dstage, 0)

        pltpu.sync_copy(shlo.at[pl.ds(srow, NPG)], slo_hbm.at[pl.ds(lo, NPG)])
        pltpu.sync_copy(shhi.at[pl.ds(srow, NPG)], shi_hbm.at[pl.ds(lo, NPG)])
        pltpu.sync_copy(accm.at[pl.ds(0, NPG)], smax_hbm.at[pl.ds(lo, NPG)])
        pltpu.sync_copy(degv.at[pl.ds(0, NPG)], deg_hbm.at[pl.ds(lo, NPG)])


BN = 400


def _tc_body(x_ref, slo_ref, shi_ref, smax_ref, deg_ref,
             w1_ref, w2a_ref, w2b_ref, w3_ref, b_ref, out_ref):
    deg = deg_ref[...]
    rdeg = 1.0 / jnp.maximum(deg, 1.0)
    mx = jnp.where(deg > 0, smax_ref[...], 0.0)
    acc = jnp.dot(x_ref[...], w1_ref[...], preferred_element_type=jnp.float32)
    acc += jnp.dot(slo_ref[...] * rdeg, w2a_ref[...],
                   preferred_element_type=jnp.float32)
    acc += jnp.dot(shi_ref[...] * rdeg, w2b_ref[...],
                   preferred_element_type=jnp.float32)
    acc += jnp.dot(mx, w3_ref[...], preferred_element_type=jnp.float32)
    out_ref[...] = jnp.maximum(acc + b_ref[...], 0.0)


_tc_linear = pl.pallas_call(
    _tc_body,
    grid=(N // BN,),
    in_specs=[
        pl.BlockSpec((BN, D), lambda i: (i, 0)),
        pl.BlockSpec((BN, DH), lambda i: (i, 0)),
        pl.BlockSpec((BN, DH), lambda i: (i, 0)),
        pl.BlockSpec((BN, D), lambda i: (i, 0)),
        pl.BlockSpec((BN, 1), lambda i: (i, 0)),
        pl.BlockSpec((D, OUT), lambda i: (0, 0)),
        pl.BlockSpec((DH, OUT), lambda i: (0, 0)),
        pl.BlockSpec((DH, OUT), lambda i: (0, 0)),
        pl.BlockSpec((D, OUT), lambda i: (0, 0)),
        pl.BlockSpec((1, OUT), lambda i: (0, 0)),
    ],
    out_specs=pl.BlockSpec((BN, OUT), lambda i: (i, 0)),
    out_shape=jax.ShapeDtypeStruct((N, OUT), jnp.float32),
)


def kernel(x, edge_index, W, b):
    src = edge_index[0]
    dst = edge_index[1]
    xlo = x[:, :DH]
    xhi = x[:, DH:]
    slo, shi, smax, deg = _sc_segment_reduce(xlo, xhi, src, dst)
    wt = W.T
    return _tc_linear(
        x, slo[:N], shi[:N], smax[:N], deg[:N].reshape(N, 1),
        wt[:D], wt[D:D + DH], wt[D + DH:2 * D], wt[2 * D:], b.reshape(1, OUT))


# speedup vs baseline: 1.8712x; 1.0476x over previous
import functools

import jax
import jax.numpy as jnp
from jax import lax
from jax.experimental import pallas as pl
from jax.experimental.pallas import tpu as pltpu
from jax.experimental.pallas import tpu_sc as plsc

N = 10000
E = 160000
D = 256
DH = 128
OUT = 256

NC = 2
NS = 16
NW = NC * NS
NPG = 160
NGRP = 2
NPAD = NW * NGRP * NPG
DUMP = NPG
SSTR = NPG + 8

CHUNK = 6400
NCHUNK = E // CHUNK
NVREG = CHUNK // 16
GB = 32
BIG = 1 << 10

_mesh = plsc.VectorSubcoreMesh(core_axis_name="c", subcore_axis_name="s")


def _treemin(v, iota16):
    for sh in (1, 2, 4, 8):
        v = jnp.minimum(v, jnp.take(v, (iota16 + sh) % 16, mode="wrap"))
    return v


def _treesum(v, iota16):
    for sh in (1, 2, 4, 8):
        v = v + jnp.take(v, (iota16 + sh) % 16, mode="wrap")
    return v


@functools.partial(
    pl.kernel,
    out_type=[
        jax.ShapeDtypeStruct((NPAD, DH), jnp.float32),
        jax.ShapeDtypeStruct((NPAD, DH), jnp.float32),
        jax.ShapeDtypeStruct((NPAD, D), jnp.float32),
        jax.ShapeDtypeStruct((NPAD,), jnp.float32),
    ],
    mesh=_mesh,
    scratch_types=[
        pltpu.VMEM((CHUNK,), jnp.int32),
        pltpu.VMEM((CHUNK,), jnp.int32),
        pltpu.VMEM((CHUNK + 3 * GB,), jnp.int32),
        pltpu.VMEM((CHUNK + 3 * GB,), jnp.int32),
        pltpu.VMEM((GB,), jnp.int32),
        pltpu.VMEM((GB,), jnp.int32),
        pltpu.VMEM((GB,), jnp.int32),
        pltpu.VMEM((GB,), jnp.int32),
        pltpu.VMEM((GB, DH), jnp.float32),
        pltpu.VMEM((GB, DH), jnp.float32),
        pltpu.VMEM((GB, DH), jnp.float32),
        pltpu.VMEM((GB, DH), jnp.float32),
        pltpu.VMEM((NPG + 1, D), jnp.float32),
        pltpu.VMEM((NPG + 16,), jnp.float32),
        pltpu.SMEM((NPG + 1,), jnp.float32),
        pltpu.VMEM_SHARED((NS * SSTR, DH), jnp.float32),
        pltpu.VMEM_SHARED((NS * SSTR, DH), jnp.float32),
        pltpu.SemaphoreType.DMA,
        pltpu.SemaphoreType.DMA,
    ],
)
def _sc_segment_reduce(xlo_hbm, xhi_hbm, src_hbm, dst_hbm,
                       slo_hbm, shi_hbm, smax_hbm, deg_hbm,
                       dstb, srcb, msrc, mdstl, gsrc_a, gsrc_b, gidx_a, gidx_b,
                       rlo_a, rhi_a, rlo_b, rhi_b, accm, degv, degs,
                       shlo, shhi, sem_a, sem_b):
    sid = lax.axis_index("s")
    wid = sid * NC + lax.axis_index("c")
    iota16 = jnp.arange(16, dtype=jnp.int32)
    zero16 = jnp.zeros((16,), jnp.float32)
    ninf16 = jnp.full((16,), -jnp.inf, jnp.float32)
    dump16 = jnp.full((16,), DUMP, jnp.int32)
    srow = sid * SSTR

    for g in range(NGRP):
        lo = (wid * NGRP + g) * NPG
        hi = lo + NPG

        def zrows(i, carry):
            rlo_a[i // 8, pl.ds((i % 8) * 16, 16)] = zero16
            return carry

        lax.fori_loop(0, GB * DH // 16, zrows, 0)
        for kk in range(NPG // GB):
            pltpu.sync_copy(rlo_a, shlo.at[pl.ds(srow + kk * GB, GB)])
            pltpu.sync_copy(rlo_a, shhi.at[pl.ds(srow + kk * GB, GB)])
        pltpu.sync_copy(rlo_a.at[pl.ds(0, 8)], shlo.at[pl.ds(srow + NPG, 8)])
        pltpu.sync_copy(rlo_a.at[pl.ds(0, 8)], shhi.at[pl.ds(srow + NPG, 8)])

        def zacc(i, carry):
            for c in range(D // 16):
                accm[i, pl.ds(c * 16, 16)] = ninf16
            return carry

        lax.fori_loop(0, NPG + 1, zacc, 0)

        def zdeg(i, carry):
            degs[i] = 0.0
            return carry

        lax.fori_loop(0, NPG + 1, zdeg, 0)

        def chunk_body(ci, carry):
            base = ci * CHUNK
            pltpu.sync_copy(dst_hbm.at[pl.ds(base, CHUNK)], dstb)
            pltpu.sync_copy(src_hbm.at[pl.ds(base, CHUNK)], srcb)

            SCB = 8

            def scan_body(bi, cnt):
                boff = bi * (16 * SCB)
                msels = []
                ktot = None
                for j in range(SCB):
                    d = dstb[pl.ds(boff + j * 16, 16)]
                    m = (d >= lo) & (d < hi)
                    msels.append(jnp.where(m, iota16 + j * 16, BIG))
                    kj = jnp.where(m, 1, 0)
                    ktot = kj if ktot is None else ktot + kj
                k = jnp.take(_treesum(ktot, iota16), iota16, mode="wrap")[0]

                def match_body(_, carry2):
                    cnt2 = carry2[0]
                    ms = list(carry2[1:])
                    comb = ms[0]
                    for j in range(1, SCB):
                        comb = jnp.minimum(comb, ms[j])
                    mnv = _treemin(comb, iota16)
                    mnd = jnp.take(mnv, iota16, mode="wrap")
                    j = mnd[0] // 16
                    voff = boff + j * 16
                    sv = srcb[pl.ds(voff, 16)]
                    dv = dstb[pl.ds(voff, 16)]
                    ssp = jnp.take(sv, mnd, mode="wrap")
                    dsp = jnp.take(dv, mnd, mode="wrap")
                    msrc[pl.ds(cnt2, 16)] = ssp
                    mdstl[pl.ds(cnt2, 16)] = dsp - lo
                    ms = [jnp.where(mj == mnv, BIG, mj) for mj in ms]
                    return (cnt2 + 1, *ms)

                out = lax.fori_loop(0, k, match_body, (cnt, *msels))
                return out[0]

            cnt = lax.fori_loop(0, NVREG // SCB, scan_body, 0)

            for i in range(3 * GB // 16):
                msrc[pl.ds(cnt + i * 16, 16)] = iota16
                mdstl[pl.ds(cnt + i * 16, 16)] = dump16

            pairs = jnp.maximum((cnt + 2 * GB - 1) // (2 * GB), 1)

            def build_and_start(si, gsrc, gidx, rlo, rhi, sem):
                sb = si * GB
                for i in range(GB // 16):
                    gsrc[pl.ds(i * 16, 16)] = msrc[pl.ds(sb + i * 16, 16)]
                    gidx[pl.ds(i * 16, 16)] = (
                        mdstl[pl.ds(sb + i * 16, 16)] + srow)
                pltpu.async_copy(xlo_hbm.at[gsrc], rlo, sem)
                pltpu.async_copy(xhi_hbm.at[gsrc], rhi, sem)

            def phase(si, gsrc, gidx, rlo, rhi, sem,
                      ngsrc, ngidx, nrlo, nrhi, nsem):
                pltpu.make_async_copy(xlo_hbm.at[gsrc], rlo, sem).wait()
                pltpu.make_async_copy(xhi_hbm.at[gsrc], rhi, sem).wait()
                build_and_start(si + 1, ngsrc, ngidx, nrlo, nrhi, nsem)
                pltpu.sync_copy(rlo, shlo.at[gidx], add=True)
                pltpu.sync_copy(rhi, shhi.at[gidx], add=True)
                sb = si * GB

                def row_body(r, carry3):
                    rdiv = r // 16
                    rm = r - rdiv * 16
                    dv = mdstl[pl.ds(sb + rdiv * 16, 16)]
                    dsp = jnp.take(dv, iota16 + rm, mode="wrap")
                    dstl = dsp[0]
                    for c in range(DH // 16):
                        rv = rlo[r, pl.ds(c * 16, 16)]
                        cm = accm[dstl, pl.ds(c * 16, 16)]
                        accm[dstl, pl.ds(c * 16, 16)] = jnp.maximum(cm, rv)
                    for c in range(DH // 16):
                        rv = rhi[r, pl.ds(c * 16, 16)]
                        cm = accm[dstl, pl.ds(DH + c * 16, 16)]
                        accm[dstl, pl.ds(DH + c * 16, 16)] = (
                            jnp.maximum(cm, rv))
                    degs[dstl] = degs[dstl] + 1.0
                    return carry3

                lax.fori_loop(0, GB, row_body, 0)

            build_and_start(0, gsrc_a, gidx_a, rlo_a, rhi_a, sem_a)

            def pair_body(p, carry2):
                phase(2 * p, gsrc_a, gidx_a, rlo_a, rhi_a, sem_a,
                      gsrc_b, gidx_b, rlo_b, rhi_b, sem_b)
                phase(2 * p + 1, gsrc_b, gidx_b, rlo_b, rhi_b, sem_b,
                      gsrc_a, gidx_a, rlo_a, rhi_a, sem_a)
                return carry2

            lax.fori_loop(0, pairs, pair_body, 0)
            pltpu.make_async_copy(xlo_hbm.at[gsrc_a], rlo_a, sem_a).wait()
            pltpu.make_async_copy(xhi_hbm.at[gsrc_a], rhi_a, sem_a).wait()
            return carry

        lax.fori_loop(0, NCHUNK, chunk_body, 0)

        def dstage(i, carry):
            degv[pl.ds(i, 16)] = zero16 + degs[i]
            return carry

        lax.fori_loop(0, NPG, dstage, 0)

        pltpu.sync_copy(shlo.at[pl.ds(srow, NPG)], slo_hbm.at[pl.ds(lo, NPG)])
        pltpu.sync_copy(shhi.at[pl.ds(srow, NPG)], shi_hbm.at[pl.ds(lo, NPG)])
        pltpu.sync_copy(accm.at[pl.ds(0, NPG)], smax_hbm.at[pl.ds(lo, NPG)])
        pltpu.sync_copy(degv.at[pl.ds(0, NPG)], deg_hbm.at[pl.ds(lo, NPG)])


BN = 400


def _tc_body(x_ref, slo_ref, shi_ref, smax_ref, deg_ref,
             w1_ref, w2a_ref, w2b_ref, w3_ref, b_ref, out_ref):
    deg = deg_ref[...]
    rdeg = 1.0 / jnp.maximum(deg, 1.0)
    mx = jnp.where(deg > 0, smax_ref[...], 0.0)
    acc = jnp.dot(x_ref[...], w1_ref[...], preferred_element_type=jnp.float32)
    acc += jnp.dot(slo_ref[...] * rdeg, w2a_ref[...],
                   preferred_element_type=jnp.float32)
    acc += jnp.dot(shi_ref[...] * rdeg, w2b_ref[...],
                   preferred_element_type=jnp.float32)
    acc += jnp.dot(mx, w3_ref[...], preferred_element_type=jnp.float32)
    out_ref[...] = jnp.maximum(acc + b_ref[...], 0.0)


_tc_linear = pl.pallas_call(
    _tc_body,
    grid=(N // BN,),
    in_specs=[
        pl.BlockSpec((BN, D), lambda i: (i, 0)),
        pl.BlockSpec((BN, DH), lambda i: (i, 0)),
        pl.BlockSpec((BN, DH), lambda i: (i, 0)),
        pl.BlockSpec((BN, D), lambda i: (i, 0)),
        pl.BlockSpec((BN, 1), lambda i: (i, 0)),
        pl.BlockSpec((D, OUT), lambda i: (0, 0)),
        pl.BlockSpec((DH, OUT), lambda i: (0, 0)),
        pl.BlockSpec((DH, OUT), lambda i: (0, 0)),
        pl.BlockSpec((D, OUT), lambda i: (0, 0)),
        pl.BlockSpec((1, OUT), lambda i: (0, 0)),
    ],
    out_specs=pl.BlockSpec((BN, OUT), lambda i: (i, 0)),
    out_shape=jax.ShapeDtypeStruct((N, OUT), jnp.float32),
)


def kernel(x, edge_index, W, b):
    src = edge_index[0]
    dst = edge_index[1]
    xlo = x[:, :DH]
    xhi = x[:, DH:]
    slo, shi, smax, deg = _sc_segment_reduce(xlo, xhi, src, dst)
    wt = W.T
    return _tc_linear(
        x, slo[:N], shi[:N], smax[:N], deg[:N].reshape(N, 1),
        wt[:D], wt[D:D + DH], wt[D + DH:2 * D], wt[2 * D:], b.reshape(1, OUT))


# speedup vs baseline: 1.9347x; 1.0339x over previous
import functools

import jax
import jax.numpy as jnp
from jax import lax
from jax.experimental import pallas as pl
from jax.experimental.pallas import tpu as pltpu
from jax.experimental.pallas import tpu_sc as plsc

N = 10000
E = 160000
D = 256
DH = 128
OUT = 256

NC = 2
NS = 16
NW = NC * NS
NPG = 160
NGRP = 2
NPAD = NW * NGRP * NPG
DUMP = NPG
SSTR = NPG + 8

CHUNK = 6400
NCHUNK = E // CHUNK
NVREG = CHUNK // 16
GB = 32
BIG = 1 << 10

_mesh = plsc.VectorSubcoreMesh(core_axis_name="c", subcore_axis_name="s")


def _treemin(v, iota16):
    for sh in (1, 2, 4, 8):
        v = jnp.minimum(v, jnp.take(v, (iota16 + sh) % 16, mode="wrap"))
    return v


def _treesum(v, iota16):
    for sh in (1, 2, 4, 8):
        v = v + jnp.take(v, (iota16 + sh) % 16, mode="wrap")
    return v


@functools.partial(
    pl.kernel,
    out_type=[
        jax.ShapeDtypeStruct((NPAD, DH), jnp.float32),
        jax.ShapeDtypeStruct((NPAD, DH), jnp.float32),
        jax.ShapeDtypeStruct((NPAD, D), jnp.float32),
        jax.ShapeDtypeStruct((NPAD,), jnp.float32),
    ],
    mesh=_mesh,
    scratch_types=[
        pltpu.VMEM((CHUNK,), jnp.int32),
        pltpu.VMEM((CHUNK,), jnp.int32),
        pltpu.VMEM((CHUNK + 3 * GB,), jnp.int32),
        pltpu.VMEM((CHUNK + 3 * GB,), jnp.int32),
        pltpu.VMEM((GB,), jnp.int32),
        pltpu.VMEM((GB,), jnp.int32),
        pltpu.VMEM((GB,), jnp.int32),
        pltpu.VMEM((GB,), jnp.int32),
        pltpu.VMEM((GB, DH), jnp.float32),
        pltpu.VMEM((GB, DH), jnp.float32),
        pltpu.VMEM((GB, DH), jnp.float32),
        pltpu.VMEM((GB, DH), jnp.float32),
        pltpu.VMEM((NPG + 1, D), jnp.float32),
        pltpu.VMEM((NPG + 16,), jnp.float32),
        pltpu.SMEM((NPG + 1,), jnp.float32),
        pltpu.VMEM_SHARED((NS * SSTR, DH), jnp.float32),
        pltpu.VMEM_SHARED((NS * SSTR, DH), jnp.float32),
        pltpu.SemaphoreType.DMA,
        pltpu.SemaphoreType.DMA,
    ],
)
def _sc_segment_reduce(xlo_hbm, xhi_hbm, src_hbm, dst_hbm,
                       slo_hbm, shi_hbm, smax_hbm, deg_hbm,
                       dstb, srcb, msrc, mdstl, gsrc_a, gsrc_b, gidx_a, gidx_b,
                       rlo_a, rhi_a, rlo_b, rhi_b, accm, degv, degs,
                       shlo, shhi, sem_a, sem_b):
    sid = lax.axis_index("s")
    wid = sid * NC + lax.axis_index("c")
    iota16 = jnp.arange(16, dtype=jnp.int32)
    zero16 = jnp.zeros((16,), jnp.float32)
    ninf16 = jnp.full((16,), -jnp.inf, jnp.float32)
    dump16 = jnp.full((16,), DUMP, jnp.int32)
    srow = sid * SSTR

    for g in range(NGRP):
        lo = (wid * NGRP + g) * NPG
        hi = lo + NPG

        def zrows(i, carry):
            rlo_a[i // 8, pl.ds((i % 8) * 16, 16)] = zero16
            return carry

        lax.fori_loop(0, GB * DH // 16, zrows, 0)
        for kk in range(NPG // GB):
            pltpu.sync_copy(rlo_a, shlo.at[pl.ds(srow + kk * GB, GB)])
            pltpu.sync_copy(rlo_a, shhi.at[pl.ds(srow + kk * GB, GB)])
        pltpu.sync_copy(rlo_a.at[pl.ds(0, 8)], shlo.at[pl.ds(srow + NPG, 8)])
        pltpu.sync_copy(rlo_a.at[pl.ds(0, 8)], shhi.at[pl.ds(srow + NPG, 8)])

        def zacc(i, carry):
            for c in range(D // 16):
                accm[i, pl.ds(c * 16, 16)] = ninf16
            return carry

        lax.fori_loop(0, NPG + 1, zacc, 0)

        def zdeg(i, carry):
            degs[i] = 0.0
            return carry

        lax.fori_loop(0, NPG + 1, zdeg, 0)

        def chunk_body(ci, carry):
            base = ci * CHUNK
            pltpu.sync_copy(dst_hbm.at[pl.ds(base, CHUNK)], dstb)
            pltpu.sync_copy(src_hbm.at[pl.ds(base, CHUNK)], srcb)

            SCB = 16

            def scan_body(bi, cnt):
                boff = bi * (16 * SCB)
                msels = []
                ktot = None
                for j in range(SCB):
                    d = dstb[pl.ds(boff + j * 16, 16)]
                    m = (d >= lo) & (d < hi)
                    msels.append(jnp.where(m, iota16 + j * 16, BIG))
                    kj = jnp.where(m, 1, 0)
                    ktot = kj if ktot is None else ktot + kj
                k = jnp.take(_treesum(ktot, iota16), iota16, mode="wrap")[0]

                def match_body(_, carry2):
                    cnt2 = carry2[0]
                    ms = list(carry2[1:])
                    comb = ms[0]
                    for j in range(1, SCB):
                        comb = jnp.minimum(comb, ms[j])
                    mnv = _treemin(comb, iota16)
                    mnd = jnp.take(mnv, iota16, mode="wrap")
                    j = mnd[0] // 16
                    voff = boff + j * 16
                    sv = srcb[pl.ds(voff, 16)]
                    dv = dstb[pl.ds(voff, 16)]
                    ssp = jnp.take(sv, mnd, mode="wrap")
                    dsp = jnp.take(dv, mnd, mode="wrap")
                    msrc[pl.ds(cnt2, 16)] = ssp
                    mdstl[pl.ds(cnt2, 16)] = dsp - lo
                    ms = [jnp.where(mj == mnv, BIG, mj) for mj in ms]
                    return (cnt2 + 1, *ms)

                out = lax.fori_loop(0, k, match_body, (cnt, *msels))
                return out[0]

            cnt = lax.fori_loop(0, NVREG // SCB, scan_body, 0)

            for i in range(3 * GB // 16):
                msrc[pl.ds(cnt + i * 16, 16)] = iota16
                mdstl[pl.ds(cnt + i * 16, 16)] = dump16

            pairs = jnp.maximum((cnt + 2 * GB - 1) // (2 * GB), 1)

            def build_and_start(si, gsrc, gidx, rlo, rhi, sem):
                sb = si * GB
                for i in range(GB // 16):
                    gsrc[pl.ds(i * 16, 16)] = msrc[pl.ds(sb + i * 16, 16)]
                    gidx[pl.ds(i * 16, 16)] = (
                        mdstl[pl.ds(sb + i * 16, 16)] + srow)
                pltpu.async_copy(xlo_hbm.at[gsrc], rlo, sem)
                pltpu.async_copy(xhi_hbm.at[gsrc], rhi, sem)

            def phase(si, gsrc, gidx, rlo, rhi, sem,
                      ngsrc, ngidx, nrlo, nrhi, nsem):
                pltpu.make_async_copy(xlo_hbm.at[gsrc], rlo, sem).wait()
                pltpu.make_async_copy(xhi_hbm.at[gsrc], rhi, sem).wait()
                build_and_start(si + 1, ngsrc, ngidx, nrlo, nrhi, nsem)
                pltpu.sync_copy(rlo, shlo.at[gidx], add=True)
                pltpu.sync_copy(rhi, shhi.at[gidx], add=True)
                sb = si * GB

                def row_body(r, carry3):
                    rdiv = r // 16
                    rm = r - rdiv * 16
                    dv = mdstl[pl.ds(sb + rdiv * 16, 16)]
                    dsp = jnp.take(dv, iota16 + rm, mode="wrap")
                    dstl = dsp[0]
                    for c in range(DH // 16):
                        rv = rlo[r, pl.ds(c * 16, 16)]
                        cm = accm[dstl, pl.ds(c * 16, 16)]
                        accm[dstl, pl.ds(c * 16, 16)] = jnp.maximum(cm, rv)
                    for c in range(DH // 16):
                        rv = rhi[r, pl.ds(c * 16, 16)]
                        cm = accm[dstl, pl.ds(DH + c * 16, 16)]
                        accm[dstl, pl.ds(DH + c * 16, 16)] = (
                            jnp.maximum(cm, rv))
                    degs[dstl] = degs[dstl] + 1.0
                    return carry3

                lax.fori_loop(0, GB, row_body, 0)

            build_and_start(0, gsrc_a, gidx_a, rlo_a, rhi_a, sem_a)

            def pair_body(p, carry2):
                phase(2 * p, gsrc_a, gidx_a, rlo_a, rhi_a, sem_a,
                      gsrc_b, gidx_b, rlo_b, rhi_b, sem_b)
                phase(2 * p + 1, gsrc_b, gidx_b, rlo_b, rhi_b, sem_b,
                      gsrc_a, gidx_a, rlo_a, rhi_a, sem_a)
                return carry2

            lax.fori_loop(0, pairs, pair_body, 0)
            pltpu.make_async_copy(xlo_hbm.at[gsrc_a], rlo_a, sem_a).wait()
            pltpu.make_async_copy(xhi_hbm.at[gsrc_a], rhi_a, sem_a).wait()
            return carry

        lax.fori_loop(0, NCHUNK, chunk_body, 0)

        def dstage(i, carry):
            degv[pl.ds(i, 16)] = zero16 + degs[i]
            return carry

        lax.fori_loop(0, NPG, dstage, 0)

        pltpu.sync_copy(shlo.at[pl.ds(srow, NPG)], slo_hbm.at[pl.ds(lo, NPG)])
        pltpu.sync_copy(shhi.at[pl.ds(srow, NPG)], shi_hbm.at[pl.ds(lo, NPG)])
        pltpu.sync_copy(accm.at[pl.ds(0, NPG)], smax_hbm.at[pl.ds(lo, NPG)])
        pltpu.sync_copy(degv.at[pl.ds(0, NPG)], deg_hbm.at[pl.ds(lo, NPG)])


BN = 400


def _tc_body(x_ref, slo_ref, shi_ref, smax_ref, deg_ref,
             w1_ref, w2a_ref, w2b_ref, w3_ref, b_ref, out_ref):
    deg = deg_ref[...]
    rdeg = 1.0 / jnp.maximum(deg, 1.0)
    mx = jnp.where(deg > 0, smax_ref[...], 0.0)
    acc = jnp.dot(x_ref[...], w1_ref[...], preferred_element_type=jnp.float32)
    acc += jnp.dot(slo_ref[...] * rdeg, w2a_ref[...],
                   preferred_element_type=jnp.float32)
    acc += jnp.dot(shi_ref[...] * rdeg, w2b_ref[...],
                   preferred_element_type=jnp.float32)
    acc += jnp.dot(mx, w3_ref[...], preferred_element_type=jnp.float32)
    out_ref[...] = jnp.maximum(acc + b_ref[...], 0.0)


_tc_linear = pl.pallas_call(
    _tc_body,
    grid=(N // BN,),
    in_specs=[
        pl.BlockSpec((BN, D), lambda i: (i, 0)),
        pl.BlockSpec((BN, DH), lambda i: (i, 0)),
        pl.BlockSpec((BN, DH), lambda i: (i, 0)),
        pl.BlockSpec((BN, D), lambda i: (i, 0)),
        pl.BlockSpec((BN, 1), lambda i: (i, 0)),
        pl.BlockSpec((D, OUT), lambda i: (0, 0)),
        pl.BlockSpec((DH, OUT), lambda i: (0, 0)),
        pl.BlockSpec((DH, OUT), lambda i: (0, 0)),
        pl.BlockSpec((D, OUT), lambda i: (0, 0)),
        pl.BlockSpec((1, OUT), lambda i: (0, 0)),
    ],
    out_specs=pl.BlockSpec((BN, OUT), lambda i: (i, 0)),
    out_shape=jax.ShapeDtypeStruct((N, OUT), jnp.float32),
)


def kernel(x, edge_index, W, b):
    src = edge_index[0]
    dst = edge_index[1]
    xlo = x[:, :DH]
    xhi = x[:, DH:]
    slo, shi, smax, deg = _sc_segment_reduce(xlo, xhi, src, dst)
    wt = W.T
    return _tc_linear(
        x, slo[:N], shi[:N], smax[:N], deg[:N].reshape(N, 1),
        wt[:D], wt[D:D + DH], wt[D + DH:2 * D], wt[2 * D:], b.reshape(1, OUT))
